# baseline (speedup 1.0000x reference)
import functools

import jax
import jax.numpy as jnp
import numpy as np
from jax import lax
from jax.experimental import pallas as pl
from jax.experimental.pallas import tpu as pltpu
from jax.experimental.pallas import tpu_sc as plsc

F32 = jnp.float32
NE, ED = 1024, 64


def _sc_gather(codebook, idx1d):
    n_rows = idx1d.shape[0]
    width = codebook.shape[1]
    nc, ns = 2, 16
    b_per_w = n_rows // (nc * ns)
    mesh = plsc.VectorSubcoreMesh(core_axis_name="c", subcore_axis_name="s")

    @functools.partial(
        pl.kernel, mesh=mesh,
        out_type=jax.ShapeDtypeStruct((n_rows, width), F32),
        scratch_types=[
            pltpu.VMEM((b_per_w,), jnp.int32),
            pltpu.VMEM((b_per_w, width), F32),
            pltpu.SemaphoreType.DMA,
        ],
    )
    def k(table_hbm, idx_hbm, out_hbm, idx_v, rows_v, sem):
        wid = lax.axis_index("s") * nc + lax.axis_index("c")
        base = wid * b_per_w
        pltpu.sync_copy(idx_hbm.at[pl.ds(base, b_per_w)], idx_v)
        pltpu.async_copy(table_hbm.at[idx_v], rows_v, sem).wait()
        pltpu.sync_copy(rows_v, out_hbm.at[pl.ds(base, b_per_w)])

    return k(codebook, idx1d)


def _enc_vq_body(x_ref, b1_ref, w1_ref, w2_ref, b2_ref, w3_ref, b3_ref,
                 cbt_ref, z_ref, idx_ref,
                 xs, xp, h1s, h1p, h2s):
    B = x_ref.shape[0]
    rows1 = B * 16 * 16
    rows2 = B * 8 ** 3

    xs[...] = jnp.zeros(xs.shape, F32)
    xs[:, 1:33, 1:33, 1:33] = x_ref[...]
    for pd in range(2):
        for ph in range(2):
            p = pd * 2 + ph
            xp[p * B:(p + 1) * B] = xs[:, pd:34:2, ph:34:2, :]

    acc1 = jnp.zeros((rows1, 512), F32)
    for kd in range(4):
        for kh in range(4):
            p = (kd % 2) * 2 + kh % 2
            jd, jh = kd // 2, kh // 2
            sl = xp[p * B:(p + 1) * B, jd:jd + 16, jh:jh + 16, :]
            acc1 = acc1 + jnp.dot(sl.reshape(rows1, 34), w1_ref[kd * 4 + kh],
                                  preferred_element_type=F32)
    h1 = jnp.maximum(acc1 + b1_ref[...], 0.0)
    h1s[...] = h1.reshape(B, 16, 16, 16, 32)

    h1p[...] = jnp.zeros(h1p.shape, F32)
    for pd in range(2):
        for ph in range(2):
            for pw in range(2):
                t = (1 - pd) * 4 + (1 - ph) * 2 + (1 - pw)
                h1p[t * B:(t + 1) * B, pd:pd + 8, ph:ph + 8, pw:pw + 8, :] = (
                    h1s[:, pd:16:2, ph:16:2, pw:16:2, :])

    def c2_body(t, acc):
        kd, kh = t // 4, t % 4
        for kw in range(4):
            p = (kd % 2) * 4 + (kh % 2) * 2 + (kw % 2)
            sl = h1p[pl.ds(p * B, B), pl.ds(kd // 2, 8), pl.ds(kh // 2, 8),
                     kw // 2:kw // 2 + 8, :]
            acc = acc + jnp.dot(sl.reshape(rows2, 32), w2_ref[t * 4 + kw],
                                preferred_element_type=F32)
        return acc
    acc2 = lax.fori_loop(0, 16, c2_body, jnp.zeros((rows2, 64), F32))
    h2 = jnp.maximum(acc2 + b2_ref[...], 0.0)
    h2s[...] = jnp.zeros(h2s.shape, F32)
    h2s[:, 1:9, 1:9, 1:9, :] = h2.reshape(B, 8, 8, 8, 64)

    def c3_body(t, acc):
        dd, dh = t // 3, t % 3
        for dw in range(3):
            sl = h2s[:, pl.ds(dd, 8), pl.ds(dh, 8), dw:dw + 8, :]
            acc = acc + jnp.dot(sl.reshape(rows2, ED), w3_ref[t * 3 + dw],
                                preferred_element_type=F32)
        return acc
    acc3 = lax.fori_loop(0, 9, c3_body, jnp.zeros((rows2, ED), F32))
    z = acc3 + b3_ref[...]
    z_ref[...] = z

    cbt = cbt_ref[...]
    c2s = jnp.sum(cbt * cbt, axis=0, keepdims=True)
    chunk = 512
    for c0 in range(0, rows2, chunk):
        zc = z[c0:c0 + chunk]
        zcb = jnp.dot(zc, cbt, preferred_element_type=F32)
        z2 = jnp.sum(zc * zc, axis=1, keepdims=True)
        d = (z2 + c2s) - 2.0 * zcb
        m = jnp.min(d, axis=1, keepdims=True)
        iota = lax.broadcasted_iota(jnp.int32, d.shape, 1)
        idx = jnp.min(jnp.where(d == m, iota, NE), axis=1, keepdims=True)
        idx_ref[c0:c0 + chunk, :] = idx


def _dec_body(zq_ref, z_ref, w4_ref, b4_ref, w5_ref, b5_ref,
              out_ref, loss_ref, zqs, gs):
    B = zqs.shape[0]
    rows2 = B * 8 ** 3

    zq = zq_ref[:, 0:ED]
    diff = zq - z_ref[...]
    s = jnp.sum(diff * diff, axis=1, keepdims=True)
    loss_ref[...] = jnp.sum(s, axis=0, keepdims=True) * (1.0 / (rows2 * ED))

    zqs[...] = jnp.zeros(zqs.shape, F32)
    zqs[:, 1:9, 1:9, 1:9, :] = zq.reshape(B, 8, 8, 8, ED)

    def d1_body(t, acc):
        dd, dh = t // 3, t % 3
        for dw in range(3):
            sl = zqs[:, pl.ds(dd, 8), pl.ds(dh, 8), dw:dw + 8, :]
            acc = acc + jnp.dot(sl.reshape(rows2, ED), w4_ref[t * 3 + dw],
                                preferred_element_type=F32)
        return acc
    acc4 = lax.fori_loop(0, 9, d1_body, jnp.zeros((rows2, 256), F32))
    g = jnp.maximum(acc4 + b4_ref[...], 0.0)
    gs[...] = jnp.zeros(gs.shape, F32)
    gs[:, 1:9, 1:9, 1:9, :] = g.reshape(B, 8, 8, 8, 256)

    def d2_body(t, acc):
        dd, dh = t // 3, t % 3
        for dw in range(3):
            sl = gs[:, pl.ds(dd, 8), pl.ds(dh, 8), dw:dw + 8, :]
            acc = acc + jnp.dot(sl.reshape(rows2, 256), w5_ref[t * 3 + dw],
                                preferred_element_type=F32)
        return acc
    acc5 = lax.fori_loop(0, 9, d2_body, jnp.zeros((rows2, 64), F32))
    out_ref[...] = acc5 + b5_ref[...]


def _sel_conv1(w1):
    S = np.zeros((4, 34, 16), np.float32)
    for kw in range(4):
        for ow in range(16):
            S[kw, 2 * ow + kw, ow] = 1.0
    w1p = w1[:, 0]
    b = jnp.einsum('kwo,cdek->dewoc', jnp.asarray(S), w1p)
    return b.reshape(16, 34, 512)


_D1 = np.zeros((3, 2, 4), np.float32)
for _a in range(3):
    for _p in range(2):
        _k = 2 * _a - _p
        if 0 <= _k < 4:
            _D1[_a, _p, _k] = 1.0

_E2 = np.zeros((3, 2, 4, 4), np.float32)
for _s in range(4):
    _u, _q = _s // 2, _s % 2
    for _d in (-1, 0, 1):
        _k = 2 * _d + 2 - _q
        if 0 <= _k < 4:
            _E2[(_u + _d) // 2 + 1, (_u + _d) % 2, _s, _k] = 1.0


def kernel(patched_tsdf, enc_w1, enc_b1, enc_w2, enc_b2, enc_w3, enc_b3,
           codebook, dec_w1, dec_b1, dec_w2, dec_b2):
    B = patched_tsdf.shape[0]

    w1r = _sel_conv1(enc_w1)
    b1r = jnp.tile(enc_b1, 16)[None, :]
    w2r = jnp.transpose(enc_w2, (2, 3, 4, 1, 0)).reshape(64, 32, 64)
    w3r = jnp.transpose(enc_w3, (2, 3, 4, 1, 0)).reshape(27, ED, ED)
    cbt = codebook.T
    d1 = jnp.asarray(_D1)
    w4r = jnp.einsum('apk,bql,crm,oiklm->abcipqro', d1, d1, d1,
                     dec_w1).reshape(27, ED, 256)
    b4r = jnp.tile(dec_b1, 8)[None, :]
    e2 = jnp.asarray(_E2)
    w5r = jnp.einsum('apsk,bqtl,crum,xklm->abcpqrxstu', e2, e2, e2,
                     dec_w2[0]).reshape(27, 256, 64)
    b5r = jnp.broadcast_to(dec_b2[None, :], (1, 64))

    z, idx = pl.pallas_call(
        _enc_vq_body,
        out_shape=(jax.ShapeDtypeStruct((B * 8 ** 3, ED), F32),
                   jax.ShapeDtypeStruct((B * 8 ** 3, 1), jnp.int32)),
        scratch_shapes=[
            pltpu.VMEM((B, 34, 34, 34), F32),
            pltpu.VMEM((4 * B, 17, 17, 34), F32),
            pltpu.VMEM((B, 16, 16, 16, 32), F32),
            pltpu.VMEM((8 * B, 9, 9, 9, 32), F32),
            pltpu.VMEM((B, 10, 10, 10, 64), F32),
        ],
    )(patched_tsdf.reshape(B, 32, 32, 32), b1r, w1r, w2r,
      enc_b2[None, :], w3r, enc_b3[None, :], cbt)

    cb128 = jnp.pad(codebook, ((0, 0), (0, 128 - ED)))
    zq = _sc_gather(cb128, idx.reshape(B * 8 ** 3))

    out5, loss = pl.pallas_call(
        _dec_body,
        out_shape=(jax.ShapeDtypeStruct((B * 8 ** 3, 64), F32),
                   jax.ShapeDtypeStruct((1, 1), F32)),
        scratch_shapes=[
            pltpu.VMEM((B, 10, 10, 10, 64), F32),
            pltpu.VMEM((B, 10, 10, 10, 256), F32),
        ],
    )(zq, z, w4r, b4r, w5r, b5r)

    loss = loss[0, 0]
    xh = out5.reshape(B, 8, 8, 8, 4, 4, 4)
    xh = jnp.transpose(xh, (0, 1, 4, 2, 5, 3, 6)).reshape(B, 1, 32, 32, 32)
    return (xh, loss, loss)
